# baseline (speedup 1.0000x reference)
import functools

import jax
import jax.numpy as jnp
from jax import lax
from jax.experimental import pallas as pl
from jax.experimental.pallas import tpu as pltpu
from jax.experimental.pallas import tpu_sc as plsc

N = 10000
E = 320000
H = 128
G = 128
OUT = 40
NUM_LAYERS = 3
_INV = 1.0 / (1.0 + 1e-5) ** 0.5

NC = 2
NS = 16
CHUNK = 128
BLK = 40
NBLK0 = 2
NBLK1 = 2
NBLK = NBLK0 + NBLK1
TCH = max(NBLK0, NBLK1) * BLK
NBUF = 2
EPAD = NS * NBLK * BLK * CHUNK
ROWS_PER_TILE = 640
NPAD = NS * ROWS_PER_TILE

_f32 = jnp.float32


def _sc_agg_body(h_hbm, src_hbm, dst_hbm, zeros_hbm, out_hbm,
                 src_v, dst_v, rows_v, acc, gsem):
    c = lax.axis_index("c")
    s = lax.axis_index("s")
    r0 = s * ROWS_PER_TILE
    if NBLK0 == NBLK1:
        nblk = NBLK0
    else:
        nblk = jnp.where(c == 0, NBLK0, NBLK1)

    pltpu.sync_copy(zeros_hbm, acc.at[pl.ds(r0, ROWS_PER_TILE)])
    plsc.subcore_barrier()

    pltpu.sync_copy(src_hbm.at[c, s], src_v)
    pltpu.sync_copy(dst_hbm.at[c, s], dst_v)

    def body(j, carry):
        pltpu.async_copy(h_hbm.at[src_v.at[j]], rows_v, gsem).wait()
        pltpu.sync_copy(rows_v, acc.at[dst_v.at[j]], add=True)
        return carry

    lax.fori_loop(0, nblk * BLK, body, 0)
    plsc.subcore_barrier()
    pltpu.sync_copy(acc.at[pl.ds(r0, ROWS_PER_TILE)],
                    out_hbm.at[pl.ds(c * NPAD + r0, ROWS_PER_TILE)])


def _sc_agg(h, src4, dst4, zrows):
    mesh = plsc.VectorSubcoreMesh(core_axis_name="c", subcore_axis_name="s")
    f = pl.kernel(
        _sc_agg_body,
        out_type=jax.ShapeDtypeStruct((NC * NPAD, H), _f32),
        mesh=mesh,
        scratch_types=[
            pltpu.VMEM((TCH, CHUNK), jnp.int32),
            pltpu.VMEM((TCH, CHUNK), jnp.int32),
            pltpu.VMEM((CHUNK, H), _f32),
            pltpu.VMEM_SHARED((NPAD, H), _f32),
            pltpu.SemaphoreType.DMA,
        ],
    )
    return f(h, src4, dst4, zrows)


def _edge_planes(idx, fill):
    pad = EPAD - E
    idx = jnp.concatenate([idx, jnp.full((pad,), fill, jnp.int32)])
    n0 = NS * NBLK0 * BLK * CHUNK
    plane = jnp.full((NC, NS, TCH, CHUNK), fill, jnp.int32)
    if NBLK0:
        plane = plane.at[0, :, 0:NBLK0 * BLK, :].set(
            idx[0:n0].reshape(NS, NBLK0 * BLK, CHUNK))
    if NBLK1:
        plane = plane.at[1, :, 0:NBLK1 * BLK, :].set(
            idx[n0:].reshape(NS, NBLK1 * BLK, CHUNK))
    return plane


def _mm(a, b):
    return jnp.dot(a, b, preferred_element_type=_f32)


def _onehots(bc, br):
    colg = lax.broadcasted_iota(jnp.int32, (N, G), 1)
    rowg = lax.broadcasted_iota(jnp.int32, (G, N), 0)
    bmem = (bc == colg).astype(_f32)
    bmem_t = (br == rowg).astype(_f32)
    return bmem, bmem_t


def _vn_mlp(pooled, vW1, vb1, vW2, vb2):
    t = jax.nn.relu(_mm(pooled, vW1) + vb1)
    return _mm(t, vW2) + vb2


def _tc0_body(x_ref, bc_ref, br_ref, Wi_ref, bi_ref, bn0w_ref, bn0b_ref,
              vne_ref, vW1_ref, vb1_ref, vW2_ref, vb2_ref,
              hpre_out, vnu_out):
    h = jax.nn.relu(_mm(x_ref[...], Wi_ref[...]) + bi_ref[...])
    h = h * (_INV * bn0w_ref[...]) + bn0b_ref[...]
    bmem, bmem_t = _onehots(bc_ref[...], br_ref[...])
    pooled = _mm(bmem_t, h) + vne_ref[...]
    vnu = _vn_mlp(pooled, vW1_ref[...], vb1_ref[...], vW2_ref[...], vb2_ref[...])
    hpre_out[...] = h + _mm(bmem, vnu)
    vnu_out[...] = vnu


def _gin_post(hpre, agg_ref, eps_ref, cW1_ref, cb1_ref, cW2_ref, cb2_ref,
              bnw_ref, bnb_ref):
    agg = agg_ref[0:N, :] + agg_ref[NPAD:NPAD + N, :]
    g = (1.0 + eps_ref[0, 0]) * hpre + agg
    g = jax.nn.relu(_mm(g, cW1_ref[...]) + cb1_ref[...])
    g = _mm(g, cW2_ref[...]) + cb2_ref[...]
    return jax.nn.relu(g * (_INV * bnw_ref[...]) + bnb_ref[...])


def _tc_mid_body(hpre_ref, agg_ref, bc_ref, br_ref, eps_ref,
                 cW1_ref, cb1_ref, cW2_ref, cb2_ref, bnw_ref, bnb_ref,
                 vprev_ref, vW1_ref, vb1_ref, vW2_ref, vb2_ref,
                 hpre_out, vnu_out):
    h = _gin_post(hpre_ref[...], agg_ref, eps_ref, cW1_ref, cb1_ref,
                  cW2_ref, cb2_ref, bnw_ref, bnb_ref)
    bmem, bmem_t = _onehots(bc_ref[...], br_ref[...])
    pooled = _mm(bmem_t, h) + vprev_ref[...]
    vnu = _vn_mlp(pooled, vW1_ref[...], vb1_ref[...], vW2_ref[...], vb2_ref[...])
    hpre_out[...] = h + _mm(bmem, vnu)
    vnu_out[...] = vnu


def _tc_fin_body(hpre_ref, agg_ref, br_ref, eps_ref,
                 cW1_ref, cb1_ref, cW2_ref, cb2_ref, bnw_ref, bnb_ref,
                 vprev_ref, f1W_ref, f1b_ref, f2W_ref, f2b_ref, out_ref):
    h = _gin_post(hpre_ref[...], agg_ref, eps_ref, cW1_ref, cb1_ref,
                  cW2_ref, cb2_ref, bnw_ref, bnb_ref)
    rowg = lax.broadcasted_iota(jnp.int32, (G, N), 0)
    bmem_t = (br_ref[...] == rowg).astype(_f32)
    ge = _mm(bmem_t, h) + vprev_ref[...]
    o = jax.nn.relu(_mm(ge, f1W_ref[...]) + f1b_ref[...])
    o = _mm(o, f2W_ref[...]) + f2b_ref[...]
    m = jnp.max(o, axis=-1, keepdims=True)
    e = jnp.exp(o - m)
    out_ref[...] = (o - m) - jnp.log(jnp.sum(e, axis=-1, keepdims=True))


_tc0 = pl.pallas_call(
    _tc0_body,
    out_shape=(jax.ShapeDtypeStruct((N, H), _f32),
               jax.ShapeDtypeStruct((G, H), _f32)),
)

_tc_mid = pl.pallas_call(
    _tc_mid_body,
    out_shape=(jax.ShapeDtypeStruct((N, H), _f32),
               jax.ShapeDtypeStruct((G, H), _f32)),
)

_tc_fin = pl.pallas_call(
    _tc_fin_body,
    out_shape=jax.ShapeDtypeStruct((G, OUT), _f32),
)


def kernel(x, edge_index, batch, W_init, b_init, bn0_w, bn0_b, vn_emb, eps,
           conv_W1, conv_b1, conv_W2, conv_b2, bn_w, bn_b,
           vn_W1, vn_b1, vn_W2, vn_b2, fc1_W, fc1_b, fc2_W, fc2_b):
    bc = batch.reshape(N, 1)
    br = batch.reshape(1, N)
    src4 = _edge_planes(edge_index[0], 0)
    dst4 = _edge_planes(edge_index[1], N)
    zrows = jnp.zeros((ROWS_PER_TILE, H), _f32)
    r2 = lambda v: v.reshape(1, -1)

    hpre, vnu = _tc0(x, bc, br, W_init, r2(b_init), r2(bn0_w), r2(bn0_b),
                     vn_emb, vn_W1[0], r2(vn_b1[0]), vn_W2[0], r2(vn_b2[0]))
    out = None
    for i in range(NUM_LAYERS):
        agg = _sc_agg(hpre, src4, dst4, zrows)
        ei = eps[i].reshape(1, 1)
        if i < NUM_LAYERS - 1:
            hpre, vnu = _tc_mid(
                hpre, agg, bc, br, ei,
                conv_W1[i], r2(conv_b1[i]), conv_W2[i], r2(conv_b2[i]),
                r2(bn_w[i]), r2(bn_b[i]),
                vnu, vn_W1[i + 1], r2(vn_b1[i + 1]), vn_W2[i + 1], r2(vn_b2[i + 1]))
        else:
            out = _tc_fin(
                hpre, agg, br, ei,
                conv_W1[i], r2(conv_b1[i]), conv_W2[i], r2(conv_b2[i]),
                r2(bn_w[i]), r2(bn_b[i]),
                vnu, fc1_W, r2(fc1_b), fc2_W, r2(fc2_b))
    return out


# speedup vs baseline: 1.4415x; 1.4415x over previous
import functools

import jax
import jax.numpy as jnp
from jax import lax
from jax.experimental import pallas as pl
from jax.experimental.pallas import tpu as pltpu
from jax.experimental.pallas import tpu_sc as plsc

N = 10000
E = 320000
H = 128
G = 128
OUT = 40
NUM_LAYERS = 3
_INV = 1.0 / (1.0 + 1e-5) ** 0.5

NC = 2
NS = 16
CHUNK = 128
CPW = 79
EPAD = NC * NS * CPW * CHUNK
ROWS_PER_TILE = 640
NPAD = NS * ROWS_PER_TILE

_f32 = jnp.float32


def _sc_agg_body(h_hbm, src_hbm, dst_hbm, zeros_hbm, out_hbm,
                 src_v, dst_v, rows_v, acc, sem):
    c = lax.axis_index("c")
    s = lax.axis_index("s")
    wid = c * NS + s
    r0 = s * ROWS_PER_TILE

    pltpu.sync_copy(zeros_hbm, acc.at[pl.ds(r0, ROWS_PER_TILE)])
    pltpu.sync_copy(src_hbm.at[wid], src_v)
    pltpu.sync_copy(dst_hbm.at[wid], dst_v)
    plsc.subcore_barrier()

    def body(j, carry):
        pltpu.async_copy(h_hbm.at[src_v.at[j]], rows_v, sem).wait()
        pltpu.sync_copy(rows_v, acc.at[dst_v.at[j]], add=True)
        return carry

    lax.fori_loop(0, CPW, body, 0)
    plsc.subcore_barrier()
    pltpu.sync_copy(acc.at[pl.ds(r0, ROWS_PER_TILE)],
                    out_hbm.at[pl.ds(c * NPAD + r0, ROWS_PER_TILE)])


def _sc_agg(h, src3, dst3, zrows):
    mesh = plsc.VectorSubcoreMesh(core_axis_name="c", subcore_axis_name="s")
    f = pl.kernel(
        _sc_agg_body,
        out_type=jax.ShapeDtypeStruct((NC * NPAD, H), _f32),
        mesh=mesh,
        scratch_types=[
            pltpu.VMEM((CPW, CHUNK), jnp.int32),
            pltpu.VMEM((CPW, CHUNK), jnp.int32),
            pltpu.VMEM((CHUNK, H), _f32),
            pltpu.VMEM_SHARED((NPAD, H), _f32),
            pltpu.SemaphoreType.DMA,
        ],
    )
    return f(h, src3, dst3, zrows)


def _mm(a, b):
    return jnp.dot(a, b, preferred_element_type=_f32)


def _onehots(bc, br):
    colg = lax.broadcasted_iota(jnp.int32, (N, G), 1)
    rowg = lax.broadcasted_iota(jnp.int32, (G, N), 0)
    bmem = (bc == colg).astype(_f32)
    bmem_t = (br == rowg).astype(_f32)
    return bmem, bmem_t


def _vn_mlp(pooled, vW1, vb1, vW2, vb2):
    t = jax.nn.relu(_mm(pooled, vW1) + vb1)
    return _mm(t, vW2) + vb2


def _tc0_body(x_ref, bc_ref, br_ref, Wi_ref, bi_ref, bn0w_ref, bn0b_ref,
              vne_ref, vW1_ref, vb1_ref, vW2_ref, vb2_ref,
              hpre_out, vnu_out):
    h = jax.nn.relu(_mm(x_ref[...], Wi_ref[...]) + bi_ref[...])
    h = h * (_INV * bn0w_ref[...]) + bn0b_ref[...]
    bmem, bmem_t = _onehots(bc_ref[...], br_ref[...])
    pooled = _mm(bmem_t, h) + vne_ref[...]
    vnu = _vn_mlp(pooled, vW1_ref[...], vb1_ref[...], vW2_ref[...], vb2_ref[...])
    hpre_out[...] = h + _mm(bmem, vnu)
    vnu_out[...] = vnu


def _gin_post(hpre, agg_ref, eps_ref, cW1_ref, cb1_ref, cW2_ref, cb2_ref,
              bnw_ref, bnb_ref):
    agg = agg_ref[0:N, :] + agg_ref[NPAD:NPAD + N, :]
    g = (1.0 + eps_ref[0, 0]) * hpre + agg
    g = jax.nn.relu(_mm(g, cW1_ref[...]) + cb1_ref[...])
    g = _mm(g, cW2_ref[...]) + cb2_ref[...]
    return jax.nn.relu(g * (_INV * bnw_ref[...]) + bnb_ref[...])


def _tc_mid_body(hpre_ref, agg_ref, bc_ref, br_ref, eps_ref,
                 cW1_ref, cb1_ref, cW2_ref, cb2_ref, bnw_ref, bnb_ref,
                 vprev_ref, vW1_ref, vb1_ref, vW2_ref, vb2_ref,
                 hpre_out, vnu_out):
    h = _gin_post(hpre_ref[...], agg_ref, eps_ref, cW1_ref, cb1_ref,
                  cW2_ref, cb2_ref, bnw_ref, bnb_ref)
    bmem, bmem_t = _onehots(bc_ref[...], br_ref[...])
    pooled = _mm(bmem_t, h) + vprev_ref[...]
    vnu = _vn_mlp(pooled, vW1_ref[...], vb1_ref[...], vW2_ref[...], vb2_ref[...])
    hpre_out[...] = h + _mm(bmem, vnu)
    vnu_out[...] = vnu


def _tc_fin_body(hpre_ref, agg_ref, br_ref, eps_ref,
                 cW1_ref, cb1_ref, cW2_ref, cb2_ref, bnw_ref, bnb_ref,
                 vprev_ref, f1W_ref, f1b_ref, f2W_ref, f2b_ref, out_ref):
    h = _gin_post(hpre_ref[...], agg_ref, eps_ref, cW1_ref, cb1_ref,
                  cW2_ref, cb2_ref, bnw_ref, bnb_ref)
    rowg = lax.broadcasted_iota(jnp.int32, (G, N), 0)
    bmem_t = (br_ref[...] == rowg).astype(_f32)
    ge = _mm(bmem_t, h) + vprev_ref[...]
    o = jax.nn.relu(_mm(ge, f1W_ref[...]) + f1b_ref[...])
    o = _mm(o, f2W_ref[...]) + f2b_ref[...]
    m = jnp.max(o, axis=-1, keepdims=True)
    e = jnp.exp(o - m)
    out_ref[...] = (o - m) - jnp.log(jnp.sum(e, axis=-1, keepdims=True))


_tc0 = pl.pallas_call(
    _tc0_body,
    out_shape=(jax.ShapeDtypeStruct((N, H), _f32),
               jax.ShapeDtypeStruct((G, H), _f32)),
)

_tc_mid = pl.pallas_call(
    _tc_mid_body,
    out_shape=(jax.ShapeDtypeStruct((N, H), _f32),
               jax.ShapeDtypeStruct((G, H), _f32)),
)

_tc_fin = pl.pallas_call(
    _tc_fin_body,
    out_shape=jax.ShapeDtypeStruct((G, OUT), _f32),
)


def kernel(x, edge_index, batch, W_init, b_init, bn0_w, bn0_b, vn_emb, eps,
           conv_W1, conv_b1, conv_W2, conv_b2, bn_w, bn_b,
           vn_W1, vn_b1, vn_W2, vn_b2, fc1_W, fc1_b, fc2_W, fc2_b):
    bc = batch.reshape(N, 1)
    br = batch.reshape(1, N)
    pad = EPAD - E
    src3 = jnp.concatenate(
        [edge_index[0], jnp.zeros((pad,), jnp.int32)]).reshape(NC * NS, CPW, CHUNK)
    dst3 = jnp.concatenate(
        [edge_index[1], jnp.full((pad,), N, jnp.int32)]).reshape(NC * NS, CPW, CHUNK)
    zrows = jnp.zeros((ROWS_PER_TILE, H), _f32)
    r2 = lambda v: v.reshape(1, -1)

    hpre, vnu = _tc0(x, bc, br, W_init, r2(b_init), r2(bn0_w), r2(bn0_b),
                     vn_emb, vn_W1[0], r2(vn_b1[0]), vn_W2[0], r2(vn_b2[0]))
    out = None
    for i in range(NUM_LAYERS):
        agg = _sc_agg(hpre, src3, dst3, zrows)
        ei = eps[i].reshape(1, 1)
        if i < NUM_LAYERS - 1:
            hpre, vnu = _tc_mid(
                hpre, agg, bc, br, ei,
                conv_W1[i], r2(conv_b1[i]), conv_W2[i], r2(conv_b2[i]),
                r2(bn_w[i]), r2(bn_b[i]),
                vnu, vn_W1[i + 1], r2(vn_b1[i + 1]), vn_W2[i + 1], r2(vn_b2[i + 1]))
        else:
            out = _tc_fin(
                hpre, agg, br, ei,
                conv_W1[i], r2(conv_b1[i]), conv_W2[i], r2(conv_b2[i]),
                r2(bn_w[i]), r2(bn_b[i]),
                vnu, fc1_W, r2(fc1_b), fc2_W, r2(fc2_b))
    return out


# speedup vs baseline: 1.4426x; 1.0008x over previous
import jax
import jax.numpy as jnp
from jax import lax
from jax.experimental import pallas as pl
from jax.experimental.pallas import tpu as pltpu
from jax.experimental.pallas import tpu_sc as plsc

N = 10000
E = 320000
H = 128
G = 128
OUT = 40
NUM_LAYERS = 3
_INV = 1.0 / (1.0 + 1e-5) ** 0.5

NC = 2
NS = 16
CHUNK = 128
CPW = 79
EPAD = NC * NS * CPW * CHUNK
ROWS_PER_TILE = 640
NPAD = NS * ROWS_PER_TILE

_f32 = jnp.float32


def _sc_agg_body(h_hbm, src_hbm, dst_hbm, zeros_hbm, out_hbm,
                 src_v, dst_v, rows_v, acc, sem):
    c = lax.axis_index("c")
    s = lax.axis_index("s")
    wid = c * NS + s
    r0 = s * ROWS_PER_TILE

    pltpu.sync_copy(zeros_hbm, acc.at[pl.ds(r0, ROWS_PER_TILE)])
    pltpu.sync_copy(src_hbm.at[wid], src_v)
    pltpu.sync_copy(dst_hbm.at[wid], dst_v)
    plsc.subcore_barrier()

    def body(j, carry):
        pltpu.async_copy(h_hbm.at[src_v.at[j]], rows_v, sem).wait()
        pltpu.sync_copy(rows_v, acc.at[dst_v.at[j]], add=True)
        return carry

    lax.fori_loop(0, CPW, body, 0)
    plsc.subcore_barrier()
    pltpu.sync_copy(acc.at[pl.ds(r0, ROWS_PER_TILE)],
                    out_hbm.at[pl.ds(c * NPAD + r0, ROWS_PER_TILE)])


def _sc_agg(h, src3, dst3, zrows):
    mesh = plsc.VectorSubcoreMesh(core_axis_name="c", subcore_axis_name="s")
    f = pl.kernel(
        _sc_agg_body,
        out_type=jax.ShapeDtypeStruct((NC * NPAD, H), _f32),
        mesh=mesh,
        scratch_types=[
            pltpu.VMEM((CPW, CHUNK), jnp.int32),
            pltpu.VMEM((CPW, CHUNK), jnp.int32),
            pltpu.VMEM((CHUNK, H), _f32),
            pltpu.VMEM_SHARED((NPAD, H), _f32),
            pltpu.SemaphoreType.DMA,
        ],
    )
    return f(h, src3, dst3, zrows)


def _mm(a, b):
    return jnp.dot(a, b, preferred_element_type=_f32)


def _onehots(bc, br):
    colg = lax.broadcasted_iota(jnp.int32, (N, G), 1)
    rowg = lax.broadcasted_iota(jnp.int32, (G, N), 0)
    bmem = (bc == colg).astype(_f32)
    bmem_t = (br == rowg).astype(_f32)
    return bmem, bmem_t


def _vn_mlp(pooled, vW1, vb1, vW2, vb2):
    t = jax.nn.relu(_mm(pooled, vW1) + vb1)
    return _mm(t, vW2) + vb2


def _tc0_body(x_ref, bc_ref, br_ref, Wi_ref, bi_ref, bn0w_ref, bn0b_ref,
              vne_ref, vW1_ref, vb1_ref, vW2_ref, vb2_ref,
              hpre_out, vnu_out):
    h = jax.nn.relu(_mm(x_ref[...], Wi_ref[...]) + bi_ref[...])
    h = h * (_INV * bn0w_ref[...]) + bn0b_ref[...]
    bmem, bmem_t = _onehots(bc_ref[...], br_ref[...])
    pooled = _mm(bmem_t, h) + vne_ref[...]
    vnu = _vn_mlp(pooled, vW1_ref[...], vb1_ref[...], vW2_ref[...], vb2_ref[...])
    hpre_out[...] = h + _mm(bmem, vnu)
    vnu_out[...] = vnu


def _gin_post(hpre, agg_ref, eps_ref, cW1_ref, cb1_ref, cW2_ref, cb2_ref,
              bnw_ref, bnb_ref):
    agg = agg_ref[0:N, :] + agg_ref[NPAD:NPAD + N, :]
    g = (1.0 + eps_ref[0, 0]) * hpre + agg
    g = jax.nn.relu(_mm(g, cW1_ref[...]) + cb1_ref[...])
    g = _mm(g, cW2_ref[...]) + cb2_ref[...]
    return jax.nn.relu(g * (_INV * bnw_ref[...]) + bnb_ref[...])


def _tc_mid_body(hpre_ref, agg_ref, bc_ref, br_ref, eps_ref,
                 cW1_ref, cb1_ref, cW2_ref, cb2_ref, bnw_ref, bnb_ref,
                 vprev_ref, vW1_ref, vb1_ref, vW2_ref, vb2_ref,
                 hpre_out, vnu_out):
    h = _gin_post(hpre_ref[...], agg_ref, eps_ref, cW1_ref, cb1_ref,
                  cW2_ref, cb2_ref, bnw_ref, bnb_ref)
    bmem, bmem_t = _onehots(bc_ref[...], br_ref[...])
    pooled = _mm(bmem_t, h) + vprev_ref[...]
    vnu = _vn_mlp(pooled, vW1_ref[...], vb1_ref[...], vW2_ref[...], vb2_ref[...])
    hpre_out[...] = h + _mm(bmem, vnu)
    vnu_out[...] = vnu


def _tc_fin_body(hpre_ref, agg_ref, br_ref, eps_ref,
                 cW1_ref, cb1_ref, cW2_ref, cb2_ref, bnw_ref, bnb_ref,
                 vprev_ref, f1W_ref, f1b_ref, f2W_ref, f2b_ref, out_ref):
    h = _gin_post(hpre_ref[...], agg_ref, eps_ref, cW1_ref, cb1_ref,
                  cW2_ref, cb2_ref, bnw_ref, bnb_ref)
    rowg = lax.broadcasted_iota(jnp.int32, (G, N), 0)
    bmem_t = (br_ref[...] == rowg).astype(_f32)
    ge = _mm(bmem_t, h) + vprev_ref[...]
    o = jax.nn.relu(_mm(ge, f1W_ref[...]) + f1b_ref[...])
    o = _mm(o, f2W_ref[...]) + f2b_ref[...]
    m = jnp.max(o, axis=-1, keepdims=True)
    e = jnp.exp(o - m)
    out_ref[...] = (o - m) - jnp.log(jnp.sum(e, axis=-1, keepdims=True))


_tc0 = pl.pallas_call(
    _tc0_body,
    out_shape=(jax.ShapeDtypeStruct((N, H), _f32),
               jax.ShapeDtypeStruct((G, H), _f32)),
)

_tc_mid = pl.pallas_call(
    _tc_mid_body,
    out_shape=(jax.ShapeDtypeStruct((N, H), _f32),
               jax.ShapeDtypeStruct((G, H), _f32)),
)

_tc_fin = pl.pallas_call(
    _tc_fin_body,
    out_shape=jax.ShapeDtypeStruct((G, OUT), _f32),
)


def kernel(x, edge_index, batch, W_init, b_init, bn0_w, bn0_b, vn_emb, eps,
           conv_W1, conv_b1, conv_W2, conv_b2, bn_w, bn_b,
           vn_W1, vn_b1, vn_W2, vn_b2, fc1_W, fc1_b, fc2_W, fc2_b):
    bc = batch.reshape(N, 1)
    br = batch.reshape(1, N)
    pad = EPAD - E
    src3 = jnp.concatenate(
        [edge_index[0], jnp.zeros((pad,), jnp.int32)]).reshape(NC * NS, CPW, CHUNK)
    dst3 = jnp.concatenate(
        [edge_index[1], jnp.full((pad,), N, jnp.int32)]).reshape(NC * NS, CPW, CHUNK)
    zrows = jnp.zeros((ROWS_PER_TILE, H), _f32)
    r2 = lambda v: v.reshape(1, -1)

    hpre, vnu = _tc0(x, bc, br, W_init, r2(b_init), r2(bn0_w), r2(bn0_b),
                     vn_emb, vn_W1[0], r2(vn_b1[0]), vn_W2[0], r2(vn_b2[0]))
    out = None
    for i in range(NUM_LAYERS):
        agg = _sc_agg(hpre, src3, dst3, zrows)
        ei = eps[i].reshape(1, 1)
        if i < NUM_LAYERS - 1:
            hpre, vnu = _tc_mid(
                hpre, agg, bc, br, ei,
                conv_W1[i], r2(conv_b1[i]), conv_W2[i], r2(conv_b2[i]),
                r2(bn_w[i]), r2(bn_b[i]),
                vnu, vn_W1[i + 1], r2(vn_b1[i + 1]), vn_W2[i + 1], r2(vn_b2[i + 1]))
        else:
            out = _tc_fin(
                hpre, agg, br, ei,
                conv_W1[i], r2(conv_b1[i]), conv_W2[i], r2(conv_b2[i]),
                r2(bn_w[i]), r2(bn_b[i]),
                vnu, fc1_W, r2(fc1_b), fc2_W, r2(fc2_b))
    return out
